# baseline (speedup 1.0000x reference)
import functools

import jax
import jax.numpy as jnp
from jax.experimental import pallas as pl
from jax.experimental.pallas import tpu as pltpu

_INPUT = 1024
_OUTPUT = 1024
_EXPERTS = 8
_TM = 256


def _moe_kernel(x_ref, gw_ref, ew_ref, eb_ref, y_ref, bl_ref, imp_ref, load_ref):
    i = pl.program_id(0)
    nsteps = pl.num_programs(0)

    @pl.when(i == 0)
    def _init():
        imp_ref[...] = jnp.zeros_like(imp_ref)
        load_ref[...] = jnp.zeros_like(load_ref)

    xt = x_ref[...]
    logits = jax.lax.dot_general(
        xt, gw_ref[...], (((1,), (1,)), ((), ())),
        preferred_element_type=jnp.float32)

    iota = jax.lax.broadcasted_iota(jnp.int32, (_TM, _EXPERTS), 1)
    m1 = jnp.max(logits, axis=1, keepdims=True)
    i1 = jnp.min(jnp.where(logits == m1, iota, _EXPERTS), axis=1, keepdims=True)
    l2 = jnp.where(iota == i1, -jnp.inf, logits)
    m2 = jnp.max(l2, axis=1, keepdims=True)
    i2 = jnp.min(jnp.where(l2 == m2, iota, _EXPERTS), axis=1, keepdims=True)

    e2 = jnp.exp(m2 - m1)
    denom = 1.0 + e2
    s1 = 1.0 / denom
    s2 = e2 / denom
    sf = jnp.where(iota == i1, s1, 0.0) + jnp.where(iota == i2, s2, 0.0)

    imp_ref[...] += jnp.sum(sf, axis=0, keepdims=True)
    load_ref[...] += jnp.sum((sf > 0.0).astype(jnp.float32), axis=0,
                             keepdims=True)

    acc = jax.lax.dot_general(
        sf, eb_ref[...], (((1,), (0,)), ((), ())),
        preferred_element_type=jnp.float32)
    xb = xt.astype(jnp.bfloat16)
    for e in range(_EXPERTS):
        pe = jax.lax.dot_general(
            xb, ew_ref[e], (((1,), (1,)), ((), ())),
            preferred_element_type=jnp.float32)
        acc = acc + sf[:, e:e + 1] * pe
    y_ref[...] = acc

    @pl.when(i == nsteps - 1)
    def _fin():
        def cv(v):
            mean = jnp.sum(v) / _EXPERTS
            var = jnp.sum((v - mean) ** 2) / (_EXPERTS - 1)
            return var / (mean * mean + 1e-10)

        bl = 0.01 * (cv(imp_ref[...]) + cv(load_ref[...]))
        bl_ref[...] = jnp.reshape(bl, (1, 1))


@functools.partial(jax.jit, static_argnames=("interpret",))
def _run(x, gate_W, expert_W, expert_b, interpret=False):
    n = x.size // x.shape[-1]
    xf = x.reshape(n, _INPUT)
    expert_W = expert_W.astype(jnp.bfloat16)
    grid = n // _TM
    y, bl = pl.pallas_call(
        _moe_kernel,
        grid=(grid,),
        in_specs=[
            pl.BlockSpec((_TM, _INPUT), lambda i: (i, 0)),
            pl.BlockSpec((_EXPERTS, _INPUT), lambda i: (0, 0)),
            pl.BlockSpec((_EXPERTS, _OUTPUT, _INPUT), lambda i: (0, 0, 0)),
            pl.BlockSpec((_EXPERTS, _OUTPUT), lambda i: (0, 0)),
        ],
        out_specs=[
            pl.BlockSpec((_TM, _OUTPUT), lambda i: (i, 0)),
            pl.BlockSpec((1, 1), lambda i: (0, 0)),
        ],
        out_shape=[
            jax.ShapeDtypeStruct((n, _OUTPUT), jnp.float32),
            jax.ShapeDtypeStruct((1, 1), jnp.float32),
        ],
        scratch_shapes=[
            pltpu.VMEM((1, _EXPERTS), jnp.float32),
            pltpu.VMEM((1, _EXPERTS), jnp.float32),
        ],
        interpret=interpret,
    )(xf, gate_W, expert_W, expert_b)
    return y.reshape(x.shape[:-1] + (_OUTPUT,)), bl[0, 0]


def kernel(x, gate_W, expert_W, expert_b):
    return _run(x, gate_W, expert_W, expert_b)


# speedup vs baseline: 1.1938x; 1.1938x over previous
import functools

import jax
import jax.numpy as jnp
from jax.experimental import pallas as pl
from jax.experimental.pallas import tpu as pltpu

_INPUT = 1024
_OUTPUT = 1024
_EXPERTS = 8
_TM = 256


def _moe_kernel(x_ref, gw_ref, ew_ref, eb_ref, y_ref, bl_ref, imp_ref, load_ref,
                wb_ref):
    i = pl.program_id(0)
    nsteps = pl.num_programs(0)

    @pl.when(i == 0)
    def _init():
        imp_ref[...] = jnp.zeros_like(imp_ref)
        load_ref[...] = jnp.zeros_like(load_ref)
        wb_ref[...] = ew_ref[...].astype(jnp.bfloat16)

    xt = x_ref[...]
    logits = jax.lax.dot_general(
        xt, gw_ref[...], (((1,), (1,)), ((), ())),
        preferred_element_type=jnp.float32)

    iota = jax.lax.broadcasted_iota(jnp.int32, (_TM, _EXPERTS), 1)
    m1 = jnp.max(logits, axis=1, keepdims=True)
    i1 = jnp.min(jnp.where(logits == m1, iota, _EXPERTS), axis=1, keepdims=True)
    l2 = jnp.where(iota == i1, -jnp.inf, logits)
    m2 = jnp.max(l2, axis=1, keepdims=True)
    i2 = jnp.min(jnp.where(l2 == m2, iota, _EXPERTS), axis=1, keepdims=True)

    e2 = jnp.exp(m2 - m1)
    denom = 1.0 + e2
    s1 = 1.0 / denom
    s2 = e2 / denom
    sf = jnp.where(iota == i1, s1, 0.0) + jnp.where(iota == i2, s2, 0.0)

    imp_ref[...] += jnp.sum(sf, axis=0, keepdims=True)
    load_ref[...] += jnp.sum((sf > 0.0).astype(jnp.float32), axis=0,
                             keepdims=True)

    acc = jax.lax.dot_general(
        sf, eb_ref[...], (((1,), (0,)), ((), ())),
        preferred_element_type=jnp.float32)
    xb = xt.astype(jnp.bfloat16)
    for e in range(_EXPERTS):
        pe = jax.lax.dot_general(
            xb, wb_ref[e], (((1,), (1,)), ((), ())),
            preferred_element_type=jnp.float32)
        acc = acc + sf[:, e:e + 1] * pe
    y_ref[...] = acc

    @pl.when(i == nsteps - 1)
    def _fin():
        def cv(v):
            mean = jnp.sum(v) / _EXPERTS
            var = jnp.sum((v - mean) ** 2) / (_EXPERTS - 1)
            return var / (mean * mean + 1e-10)

        bl = 0.01 * (cv(imp_ref[...]) + cv(load_ref[...]))
        bl_ref[...] = jnp.reshape(bl, (1, 1))


@functools.partial(jax.jit, static_argnames=("interpret",))
def _run(x, gate_W, expert_W, expert_b, interpret=False):
    n = x.size // x.shape[-1]
    xf = x.reshape(n, _INPUT)
    grid = n // _TM
    y, bl = pl.pallas_call(
        _moe_kernel,
        grid=(grid,),
        in_specs=[
            pl.BlockSpec((_TM, _INPUT), lambda i: (i, 0)),
            pl.BlockSpec((_EXPERTS, _INPUT), lambda i: (0, 0)),
            pl.BlockSpec((_EXPERTS, _OUTPUT, _INPUT), lambda i: (0, 0, 0)),
            pl.BlockSpec((_EXPERTS, _OUTPUT), lambda i: (0, 0)),
        ],
        out_specs=[
            pl.BlockSpec((_TM, _OUTPUT), lambda i: (i, 0)),
            pl.BlockSpec((1, 1), lambda i: (0, 0)),
        ],
        out_shape=[
            jax.ShapeDtypeStruct((n, _OUTPUT), jnp.float32),
            jax.ShapeDtypeStruct((1, 1), jnp.float32),
        ],
        scratch_shapes=[
            pltpu.VMEM((1, _EXPERTS), jnp.float32),
            pltpu.VMEM((1, _EXPERTS), jnp.float32),
            pltpu.VMEM((_EXPERTS, _OUTPUT, _INPUT), jnp.bfloat16),
        ],
        interpret=interpret,
    )(xf, gate_W, expert_W, expert_b)
    return y.reshape(x.shape[:-1] + (_OUTPUT,)), bl[0, 0]


def kernel(x, gate_W, expert_W, expert_b):
    return _run(x, gate_W, expert_W, expert_b)


# speedup vs baseline: 1.2899x; 1.0805x over previous
import functools

import jax
import jax.numpy as jnp
from jax.experimental import pallas as pl
from jax.experimental.pallas import tpu as pltpu

_INPUT = 1024
_OUTPUT = 1024
_EXPERTS = 8


def _moe_kernel(x_ref, gw_ref, ew_ref, eb_ref, y_ref, bl_ref, sf_ref):
    e = pl.program_id(0)
    n = x_ref.shape[0]

    @pl.when(e == 0)
    def _gate():
        xt = x_ref[...]
        logits = jax.lax.dot_general(
            xt, gw_ref[...], (((1,), (1,)), ((), ())),
            preferred_element_type=jnp.float32)
        iota = jax.lax.broadcasted_iota(jnp.int32, (n, _EXPERTS), 1)
        m1 = jnp.max(logits, axis=1, keepdims=True)
        i1 = jnp.min(jnp.where(logits == m1, iota, _EXPERTS), axis=1,
                     keepdims=True)
        l2 = jnp.where(iota == i1, -jnp.inf, logits)
        m2 = jnp.max(l2, axis=1, keepdims=True)
        i2 = jnp.min(jnp.where(l2 == m2, iota, _EXPERTS), axis=1,
                     keepdims=True)
        ex = jnp.exp(m2 - m1)
        denom = 1.0 + ex
        s1 = 1.0 / denom
        s2 = ex / denom
        sf = jnp.where(iota == i1, s1, 0.0) + jnp.where(iota == i2, s2, 0.0)
        sf_ref[...] = sf

        def cv(v):
            mean = jnp.sum(v) / _EXPERTS
            var = jnp.sum((v - mean) ** 2) / (_EXPERTS - 1)
            return var / (mean * mean + 1e-10)

        imp = jnp.sum(sf, axis=0)
        load = jnp.sum((sf > 0.0).astype(jnp.float32), axis=0)
        bl_ref[...] = jnp.reshape(0.01 * (cv(imp) + cv(load)), (1, 1))

        y_ref[...] = jax.lax.dot_general(
            sf, eb_ref[...], (((1,), (0,)), ((), ())),
            preferred_element_type=jnp.float32)

    xb = x_ref[...].astype(jnp.bfloat16)
    pe = jax.lax.dot_general(
        xb, ew_ref[0], (((1,), (1,)), ((), ())),
        preferred_element_type=jnp.float32)
    iota = jax.lax.broadcasted_iota(jnp.int32, (n, _EXPERTS), 1)
    sf_col = jnp.sum(jnp.where(iota == e, sf_ref[...], 0.0), axis=1,
                     keepdims=True)
    y_ref[...] += sf_col * pe


@functools.partial(jax.jit, static_argnames=("interpret",))
def _run(x, gate_W, expert_W, expert_b, interpret=False):
    n = x.size // x.shape[-1]
    xf = x.reshape(n, _INPUT)
    y, bl = pl.pallas_call(
        _moe_kernel,
        grid=(_EXPERTS,),
        in_specs=[
            pl.BlockSpec((n, _INPUT), lambda e: (0, 0)),
            pl.BlockSpec((_EXPERTS, _INPUT), lambda e: (0, 0)),
            pl.BlockSpec((1, _OUTPUT, _INPUT), lambda e: (e, 0, 0)),
            pl.BlockSpec((_EXPERTS, _OUTPUT), lambda e: (0, 0)),
        ],
        out_specs=[
            pl.BlockSpec((n, _OUTPUT), lambda e: (0, 0)),
            pl.BlockSpec((1, 1), lambda e: (0, 0)),
        ],
        out_shape=[
            jax.ShapeDtypeStruct((n, _OUTPUT), jnp.float32),
            jax.ShapeDtypeStruct((1, 1), jnp.float32),
        ],
        scratch_shapes=[
            pltpu.VMEM((n, _EXPERTS), jnp.float32),
        ],
        interpret=interpret,
    )(xf, gate_W, expert_W, expert_b)
    return y.reshape(x.shape[:-1] + (_OUTPUT,)), bl[0, 0]


def kernel(x, gate_W, expert_W, expert_b):
    return _run(x, gate_W, expert_W, expert_b)
